# Initial kernel scaffold; baseline (speedup 1.0000x reference)
import jax
import jax.numpy as jnp
from jax.experimental import pallas as pl


def kernel(input, embeddings):
    raise NotImplementedError("write your pallas kernel here")


# speedup vs baseline: 4.8538x; 4.8538x over previous
import functools

import jax
import jax.numpy as jnp
from jax import lax
from jax.experimental import pallas as pl
from jax.experimental.pallas import tpu as pltpu
from jax.experimental.pallas import tpu_sc as plsc

_NC = 2
_NS = 16
_NW = _NC * _NS
_CHUNK = 512
_GRP = 128


def _gather_flat(idx_flat, table):
    B = idx_flat.shape[0]
    D = table.shape[1]
    b_per_w = B // _NW
    n_chunks = b_per_w // _CHUNK
    mesh = plsc.VectorSubcoreMesh(core_axis_name="c", subcore_axis_name="s")

    @functools.partial(
        pl.kernel,
        out_type=jax.ShapeDtypeStruct((B, D), jnp.float32),
        mesh=mesh,
        scratch_types=[
            pltpu.VMEM((_CHUNK,), jnp.int32),
            pltpu.VMEM((_CHUNK, D), jnp.float32),
            pltpu.SemaphoreType.DMA,
        ],
        compiler_params=pltpu.CompilerParams(use_tc_tiling_on_sc=False),
    )
    def k(idx_hbm, table_hbm, out_hbm, idx_v, rows_v, sem):
        wid = lax.axis_index("s") * _NC + lax.axis_index("c")
        base_w = wid * b_per_w

        def chunk_body(i, carry):
            base = base_w + i * _CHUNK
            pltpu.sync_copy(idx_hbm.at[pl.ds(base, _CHUNK)], idx_v)
            cps = []
            for g in range(_CHUNK // _GRP):
                cps.append(pltpu.async_copy(
                    table_hbm.at[idx_v.at[pl.ds(g * _GRP, _GRP)]],
                    rows_v.at[pl.ds(g * _GRP, _GRP)],
                    sem))
            for cp in cps:
                cp.wait()
            pltpu.sync_copy(rows_v, out_hbm.at[pl.ds(base, _CHUNK)])
            return carry

        lax.fori_loop(0, n_chunks, chunk_body, 0)

    return k(idx_flat, table)


def kernel(input, embeddings):
    seq_a, seq_b = input.shape
    idx_flat = input.reshape(-1).astype(jnp.int32)
    out = _gather_flat(idx_flat, embeddings)
    return out.reshape(seq_a, seq_b, embeddings.shape[1])


# speedup vs baseline: 5.1446x; 1.0599x over previous
import functools

import jax
import jax.numpy as jnp
from jax import lax
from jax.experimental import pallas as pl
from jax.experimental.pallas import tpu as pltpu
from jax.experimental.pallas import tpu_sc as plsc

_NC = 2
_NS = 16
_NW = _NC * _NS
_NBUF = 2
_CHUNK = 512
_GRP = 128
_NGRP = _CHUNK // _GRP


def _gather_flat(idx_flat, table):
    B = idx_flat.shape[0]
    D = table.shape[1]
    b_per_w = B // _NW
    n_chunks = b_per_w // _CHUNK
    n_outer = n_chunks // _NBUF
    assert B % _NW == 0 and b_per_w % _CHUNK == 0
    assert n_chunks % _NBUF == 0 and n_outer >= 3
    mesh = plsc.VectorSubcoreMesh(core_axis_name="c", subcore_axis_name="s")

    @functools.partial(
        pl.kernel,
        out_type=jax.ShapeDtypeStruct((B, D), jnp.float32),
        mesh=mesh,
        scratch_types=(
            [pltpu.VMEM((_NBUF, _CHUNK), jnp.int32),
             pltpu.VMEM((_NBUF, _CHUNK, D), jnp.float32)]
            + [pltpu.SemaphoreType.DMA] * (3 * _NBUF)
        ),
        compiler_params=pltpu.CompilerParams(use_tc_tiling_on_sc=False),
    )
    def k(idx_hbm, table_hbm, out_hbm, idx_v, rows_v, *sems):
        sem_idx = sems[0:_NBUF]
        sem_rows = sems[_NBUF:2 * _NBUF]
        sem_out = sems[2 * _NBUF:3 * _NBUF]
        wid = lax.axis_index("s") * _NC + lax.axis_index("c")
        base_w = wid * b_per_w

        def issue_idx(i, b):
            pltpu.async_copy(
                idx_hbm.at[pl.ds(base_w + i * _CHUNK, _CHUNK)],
                idx_v.at[b], sem_idx[b])

        def wait_idx(b):
            pltpu.make_async_copy(
                idx_hbm.at[pl.ds(base_w, _CHUNK)],
                idx_v.at[b], sem_idx[b]).wait()

        def issue_gathers(b):
            for g in range(_NGRP):
                pltpu.async_copy(
                    table_hbm.at[idx_v.at[b, pl.ds(g * _GRP, _GRP)]],
                    rows_v.at[b, pl.ds(g * _GRP, _GRP)],
                    sem_rows[b])

        def wait_gathers(b):
            for g in range(_NGRP):
                pltpu.make_async_copy(
                    table_hbm.at[idx_v.at[b, pl.ds(g * _GRP, _GRP)]],
                    rows_v.at[b, pl.ds(g * _GRP, _GRP)],
                    sem_rows[b]).wait()

        def issue_store(i, b):
            pltpu.async_copy(
                rows_v.at[b],
                out_hbm.at[pl.ds(base_w + i * _CHUNK, _CHUNK)],
                sem_out[b])

        def wait_store(b):
            pltpu.make_async_copy(
                rows_v.at[b],
                out_hbm.at[pl.ds(base_w, _CHUNK)],
                sem_out[b]).wait()

        def chunk_step(i, b, do_store_wait, do_prev, do_idx_issue):
            if do_store_wait:
                wait_store(b)
            wait_idx(b)
            issue_gathers(b)
            if do_prev:
                pb = (b - 1) % _NBUF
                wait_gathers(pb)
                issue_store(i - 1, pb)
                if do_idx_issue:
                    issue_idx(i - 1 + _NBUF, pb)

        for b in range(_NBUF):
            issue_idx(b, b)
        chunk_step(0, 0, False, False, False)
        for b in range(1, _NBUF):
            chunk_step(b, b, False, True, True)

        def outer_body(g, carry):
            for b in range(_NBUF):
                chunk_step(g * _NBUF + b, b, True, True, True)
            return carry

        lax.fori_loop(1, n_outer - 1, outer_body, 0)

        i0 = (n_outer - 1) * _NBUF
        for b in range(_NBUF - 1):
            chunk_step(i0 + b, b, True, True, True)
        chunk_step(n_chunks - 1, _NBUF - 1, True, True, False)

        last = (_NBUF - 1) % _NBUF
        wait_gathers(last)
        issue_store(n_chunks - 1, last)
        for b in range(_NBUF):
            wait_store(b)

    return k(idx_flat, table)


def kernel(input, embeddings):
    seq_a, seq_b = input.shape
    idx_flat = input.reshape(-1).astype(jnp.int32)
    out = _gather_flat(idx_flat, embeddings)
    return out.reshape(seq_a, seq_b, embeddings.shape[1])


# speedup vs baseline: 6.2448x; 1.2139x over previous
import functools

import jax
import jax.numpy as jnp
from jax import lax
from jax.experimental import pallas as pl
from jax.experimental.pallas import tpu as pltpu
from jax.experimental.pallas import tpu_sc as plsc

_NC = 2
_NS = 16
_SEQ = 2048
_D = 64
_KPW = 8
_JW = 1024
_ROWS_PER_CORE = _SEQ // _NC


def _gather_t(idx, table_t):
    mesh = plsc.VectorSubcoreMesh(core_axis_name="c", subcore_axis_name="s")

    @functools.partial(
        pl.kernel,
        out_type=jax.ShapeDtypeStruct((_SEQ, _D, _SEQ), jnp.float32),
        mesh=mesh,
        scratch_types=(
            [pltpu.VMEM((_KPW, 4097), jnp.float32)]
            + [pltpu.VMEM((_JW,), jnp.int32)] * 2
            + [pltpu.VMEM((1, _KPW, _JW), jnp.float32)] * 2
            + [pltpu.SemaphoreType.DMA] * 4
        ),
        compiler_params=pltpu.CompilerParams(needs_layout_passes=False),
    )
    def k(idx_hbm, tab_hbm, out_hbm, tabv, idxv0, idxv1, outv0, outv1,
          sem_i0, sem_i1, sem_o0, sem_o1):
        idxv = (idxv0, idxv1)
        outv = (outv0, outv1)
        sem_idx = (sem_i0, sem_i1)
        sem_out = (sem_o0, sem_o1)
        c = lax.axis_index("c")
        s = lax.axis_index("s")
        k0 = (s % 8) * _KPW
        j0 = (s // 8) * _JW
        row0 = c * _ROWS_PER_CORE

        pltpu.sync_copy(tab_hbm.at[pl.ds(k0, _KPW)], tabv)

        def issue_idx(i, b):
            pltpu.async_copy(
                idx_hbm.at[pl.ds((row0 + i) * _SEQ + j0, _JW)],
                idxv[b], sem_idx[b])

        def wait_idx(b):
            pltpu.make_async_copy(
                idx_hbm.at[pl.ds(0, _JW)],
                idxv[b], sem_idx[b]).wait()

        def issue_store(i, b):
            pltpu.async_copy(
                outv[b],
                out_hbm.at[pl.ds(row0 + i, 1), pl.ds(k0, _KPW), pl.ds(j0, _JW)],
                sem_out[b])

        def wait_store(b):
            pltpu.make_async_copy(
                outv[b],
                out_hbm.at[pl.ds(0, 1), pl.ds(k0, _KPW), pl.ds(j0, _JW)],
                sem_out[b]).wait()

        def compute(b):
            def jbody(j16, carry):
                iv = idxv[b][pl.ds(j16 * 16, 16)]
                for kr in range(_KPW):
                    kv = jnp.full((16,), kr, jnp.int32)
                    outv[b][0, kr, pl.ds(j16 * 16, 16)] = (
                        plsc.load_gather(tabv, [kv, iv]))
                return carry
            lax.fori_loop(0, _JW // 16, jbody, 0)

        issue_idx(0, 0)
        issue_idx(1, 1)

        def body(i, b):
            wait_idx(b)

            @pl.when(i >= 2)
            def _():
                wait_store(b)

            compute(b)
            issue_store(i, b)

            @pl.when(i + 2 < _ROWS_PER_CORE)
            def _():
                issue_idx(i + 2, b)

        def outer(g, carry):
            body(g * 2, 0)
            body(g * 2 + 1, 1)
            return carry

        lax.fori_loop(0, _ROWS_PER_CORE // 2, outer, 0)
        wait_store(0)
        wait_store(1)

    return k(idx, table_t)


def kernel(input, embeddings):
    table_t = jnp.swapaxes(embeddings, 0, 1)
    out = _gather_t(input.reshape(-1).astype(jnp.int32), table_t)
    return jnp.swapaxes(out, 1, 2)


# speedup vs baseline: 22.1507x; 3.5471x over previous
import functools

import jax
import jax.numpy as jnp
from jax import lax
from jax.experimental import pallas as pl
from jax.experimental.pallas import tpu as pltpu
from jax.experimental.pallas import tpu_sc as plsc

_NC = 2
_NS = 16
_SEQ = 2048
_D = 64
_KPW = 8
_JW = 1024
_ROWS_PER_CORE = _SEQ // _NC


def _gather_t(idx, table_t):
    mesh = plsc.VectorSubcoreMesh(core_axis_name="c", subcore_axis_name="s")

    @functools.partial(
        pl.kernel,
        out_type=jax.ShapeDtypeStruct((_SEQ, _D, _SEQ), jnp.float32),
        mesh=mesh,
        scratch_types=(
            [pltpu.VMEM((_KPW, 4097), jnp.float32)]
            + [pltpu.VMEM((_JW,), jnp.int32)] * 2
            + [pltpu.VMEM((1, _KPW, _JW), jnp.float32)] * 2
            + [pltpu.SemaphoreType.DMA] * 4
        ),
        compiler_params=pltpu.CompilerParams(needs_layout_passes=False),
    )
    def k(idx_hbm, tab_hbm, out_hbm, tabv, idxv0, idxv1, outv0, outv1,
          sem_i0, sem_i1, sem_o0, sem_o1):
        idxv = (idxv0, idxv1)
        outv = (outv0, outv1)
        sem_idx = (sem_i0, sem_i1)
        sem_out = (sem_o0, sem_o1)
        c = lax.axis_index("c")
        s = lax.axis_index("s")
        k0 = (s % 8) * _KPW
        j0 = (s // 8) * _JW
        row0 = c * _ROWS_PER_CORE

        pltpu.sync_copy(tab_hbm.at[pl.ds(k0, _KPW)], tabv)

        def issue_idx(i, b):
            pltpu.async_copy(
                idx_hbm.at[pl.ds((row0 + i) * _SEQ + j0, _JW)],
                idxv[b], sem_idx[b])

        def wait_idx(b):
            pltpu.make_async_copy(
                idx_hbm.at[pl.ds(0, _JW)],
                idxv[b], sem_idx[b]).wait()

        def issue_store(i, b):
            pltpu.async_copy(
                outv[b],
                out_hbm.at[pl.ds(row0 + i, 1), pl.ds(k0, _KPW), pl.ds(j0, _JW)],
                sem_out[b])

        def wait_store(b):
            pltpu.make_async_copy(
                outv[b],
                out_hbm.at[pl.ds(0, 1), pl.ds(k0, _KPW), pl.ds(j0, _JW)],
                sem_out[b]).wait()

        def compute(b):
            @plsc.parallel_loop(0, _JW // 16, unroll=4)
            def _(j16):
                iv = idxv[b][pl.ds(j16 * 16, 16)]
                vals = [plsc.load_gather(
                            tabv, [jnp.full((16,), kr, jnp.int32), iv])
                        for kr in range(_KPW)]
                for kr in range(_KPW):
                    outv[b][0, kr, pl.ds(j16 * 16, 16)] = vals[kr]

        issue_idx(0, 0)
        issue_idx(1, 1)

        def body(i, b):
            wait_idx(b)

            @pl.when(i >= 2)
            def _():
                wait_store(b)

            compute(b)
            issue_store(i, b)

            @pl.when(i + 2 < _ROWS_PER_CORE)
            def _():
                issue_idx(i + 2, b)

        def outer(g, carry):
            body(g * 2, 0)
            body(g * 2 + 1, 1)
            return carry

        lax.fori_loop(0, _ROWS_PER_CORE // 2, outer, 0)
        wait_store(0)
        wait_store(1)

    return k(idx, table_t)


def kernel(input, embeddings):
    table_t = jnp.swapaxes(embeddings, 0, 1)
    out = _gather_t(input.reshape(-1).astype(jnp.int32), table_t)
    return jnp.swapaxes(out, 1, 2)
